# baseline (speedup 1.0000x reference)
import jax
import jax.numpy as jnp
from jax import lax
from jax.experimental import pallas as pl
from jax.experimental.pallas import tpu as pltpu, tpu_sc as plsc

N_NODES = 10000
N_EDGES = 320000
D = 128
DH = D // 2
N_REL = 100

NC = 2
NS = 16
E_PER_T = N_EDGES // NS
CHUNK = 80
N_CHUNKS = E_PER_T // CHUNK
ROWS_MAIN = 624

_mesh = plsc.VectorSubcoreMesh(
    core_axis_name="c", subcore_axis_name="s", num_cores=NC, num_subcores=NS)


def _gnn_body(ha, hb, ra0, rb0, ra1, rb1, src1, dst1, et1, outa, outb,
              rows0, rows1, rows2, relr0, relr1, relr2,
              sb0, sb1, sb2, eb0, eb1, eb2, db0, db1, db2,
              acc1, acc2, rel_sp0, rel_sp1,
              sgh0, sgh1, sgh2, sgr0, sgr1, sgr2,
              sse0, sse1, sse2, sd0, sd1, sd2, ss0, ss1, ss2):
  cid = lax.axis_index("c")
  sid = lax.axis_index("s")

  @pl.when(jnp.logical_and(cid == 0, sid == 0))
  def _():
    pltpu.sync_copy(ra0, rel_sp0)
    pltpu.sync_copy(ra1, rel_sp1)

  @pl.when(jnp.logical_and(cid == 1, sid == 0))
  def _():
    pltpu.sync_copy(rb0, rel_sp0)
    pltpu.sync_copy(rb1, rel_sp1)

  zv = jnp.zeros((16,), jnp.float32)

  @plsc.parallel_loop(0, CHUNK)
  def _(r):
    for c in range(DH // 16):
      rows0[r, pl.ds(c * 16, 16)] = zv

  row0 = sid * ROWS_MAIN
  tail = N_NODES - NS * ROWS_MAIN
  for acc in (acc1, acc2):
    for k in range(ROWS_MAIN // CHUNK):
      pltpu.sync_copy(rows0, acc.at[pl.ds(row0 + k * CHUNK, CHUNK)])
    rem = ROWS_MAIN % CHUNK
    pltpu.sync_copy(rows0.at[pl.ds(0, rem)],
                    acc.at[pl.ds(row0 + ROWS_MAIN - rem, rem)])

    @pl.when(sid == NS - 1)
    def _():
      pltpu.sync_copy(rows0.at[pl.ds(0, tail)],
                      acc.at[pl.ds(NS * ROWS_MAIN, tail)])

  plsc.subcore_barrier()

  base_e = sid * E_PER_T

  bufs = ((rows0, relr0, sb0, eb0, db0, sgh0, sgr0, sse0, sd0, ss0),
          (rows1, relr1, sb1, eb1, db1, sgh1, sgr1, sse1, sd1, ss1),
          (rows2, relr2, sb2, eb2, db2, sgh2, sgr2, sse2, sd2, ss2))

  def refill_se(i, b):
    sb, eb, sse = bufs[b][2], bufs[b][3], bufs[b][7]
    pltpu.async_copy(src1.at[pl.ds(base_e + i * CHUNK, CHUNK)], sb, sse)
    pltpu.async_copy(et1.at[pl.ds(base_e + i * CHUNK, CHUNK)], eb, sse)

  def wait_se(i, b):
    sb, eb, sse = bufs[b][2], bufs[b][3], bufs[b][7]
    pltpu.make_async_copy(src1.at[pl.ds(base_e + i * CHUNK, CHUNK)], sb,
                          sse).wait()
    pltpu.make_async_copy(et1.at[pl.ds(base_e + i * CHUNK, CHUNK)], eb,
                          sse).wait()

  def refill_d(i, b):
    db, sd = bufs[b][4], bufs[b][8]
    pltpu.async_copy(dst1.at[pl.ds(base_e + i * CHUNK, CHUNK)], db, sd)

  def wait_d(i, b):
    db, sd = bufs[b][4], bufs[b][8]
    pltpu.make_async_copy(dst1.at[pl.ds(base_e + i * CHUNK, CHUNK)], db,
                          sd).wait()

  def mul(b):
    rows, relr = bufs[b][0], bufs[b][1]

    @plsc.parallel_loop(0, CHUNK, unroll=8)
    def _(r):
      for c in range(DH // 16):
        s = pl.ds(c * 16, 16)
        rows[r, s] = rows[r, s] * relr[r, s]

  def make_phase(src_tab, rel_sp, acc):
    per_core = isinstance(src_tab, tuple)

    def issue_gather(b):
      rows, relr = bufs[b][0], bufs[b][1]
      sb, eb = bufs[b][2], bufs[b][3]
      sgh, sgr = bufs[b][5], bufs[b][6]
      if per_core:
        @pl.when(cid == 0)
        def _():
          pltpu.async_copy(src_tab[0].at[sb], rows, sgh)

        @pl.when(cid == 1)
        def _():
          pltpu.async_copy(src_tab[1].at[sb], rows, sgh)
      else:
        pltpu.async_copy(src_tab.at[sb], rows, sgh)
      pltpu.async_copy(rel_sp.at[eb], relr, sgr)

    def wait_gather(b):
      rows, relr = bufs[b][0], bufs[b][1]
      sb, eb = bufs[b][2], bufs[b][3]
      sgh, sgr = bufs[b][5], bufs[b][6]
      ref = src_tab[0] if per_core else src_tab
      pltpu.make_async_copy(ref.at[sb], rows, sgh).wait()
      pltpu.make_async_copy(rel_sp.at[eb], relr, sgr).wait()

    def issue_scatter(b):
      rows, db, ss = bufs[b][0], bufs[b][4], bufs[b][9]
      pltpu.async_copy(rows, acc.at[db], ss, add=True)

    def wait_scatter(b):
      rows, db, ss = bufs[b][0], bufs[b][4], bufs[b][9]
      pltpu.make_async_copy(rows, acc.at[db], ss).wait()

    def section(i, b, bn, bnn):
      @pl.when(i >= 2)
      def _():
        wait_scatter(bn)

      @pl.when(i + 1 < N_CHUNKS)
      def _():
        refill_d(i + 1, bn)
        wait_se(i + 1, bn)
        issue_gather(bn)

      @pl.when(i + 2 < N_CHUNKS)
      def _():
        refill_se(i + 2, bnn)

      wait_gather(b)
      mul(b)
      wait_d(i, b)
      issue_scatter(b)

    def run():
      refill_se(0, 0)
      refill_se(1, 1)
      refill_d(0, 0)
      wait_se(0, 0)
      issue_gather(0)

      def _trio(g, _):
        for k in range(3):
          section(3 * g + k, k, (k + 1) % 3, (k + 2) % 3)
        return _

      n_trios = N_CHUNKS // 3
      lax.fori_loop(0, n_trios, _trio, None)
      for i in range(3 * n_trios, N_CHUNKS):
        section(jnp.int32(i), i % 3, (i + 1) % 3, (i + 2) % 3)
      wait_scatter((N_CHUNKS - 2) % 3)
      wait_scatter((N_CHUNKS - 1) % 3)

    return run

  make_phase((ha, hb), rel_sp0, acc1)()
  plsc.subcore_barrier()
  make_phase(acc1, rel_sp1, acc2)()
  plsc.subcore_barrier()

  for c, out in ((0, outa), (1, outb)):

    @pl.when(cid == c)
    def _():
      pltpu.sync_copy(acc2.at[pl.ds(row0, ROWS_MAIN)],
                      out.at[pl.ds(row0, ROWS_MAIN)])

      @pl.when(sid == NS - 1)
      def _():
        pltpu.sync_copy(acc2.at[pl.ds(NS * ROWS_MAIN, tail)],
                        out.at[pl.ds(NS * ROWS_MAIN, tail)])


_gnn2 = pl.kernel(
    _gnn_body,
    out_type=(jax.ShapeDtypeStruct((N_NODES, DH), jnp.float32),
              jax.ShapeDtypeStruct((N_NODES, DH), jnp.float32)),
    mesh=_mesh,
    compiler_params=pltpu.CompilerParams(use_tc_tiling_on_sc=False),
    scratch_types=(
        [pltpu.VMEM((CHUNK, DH), jnp.float32)] * 6
        + [pltpu.VMEM((CHUNK,), jnp.int32)] * 9
        + [pltpu.VMEM_SHARED((N_NODES, DH), jnp.float32)] * 2
        + [pltpu.VMEM_SHARED((N_REL, DH), jnp.float32)] * 2
        + [pltpu.SemaphoreType.DMA] * 15
    ),
)


def kernel(node_features, edge_index, etype, rel_emb_0, rel_emb_1):
  src = edge_index[0].astype(jnp.int32)
  dst = edge_index[1].astype(jnp.int32)
  et = etype.astype(jnp.int32)
  ha = node_features[:, :DH]
  hb = node_features[:, DH:]
  oa, ob = _gnn2(ha, hb,
                 rel_emb_0[:, :DH], rel_emb_0[:, DH:],
                 rel_emb_1[:, :DH], rel_emb_1[:, DH:],
                 src, dst, et)
  return jnp.concatenate([oa, ob], axis=1)


# speedup vs baseline: 1.1442x; 1.1442x over previous
import jax
import jax.numpy as jnp
from jax import lax
from jax.experimental import pallas as pl
from jax.experimental.pallas import tpu as pltpu, tpu_sc as plsc

N_NODES = 10000
N_EDGES = 320000
D = 128
DH = D // 2
N_REL = 100

NC = 2
NS = 16
E_PER_T = N_EDGES // NS
CHUNK = 80
N_CHUNKS = E_PER_T // CHUNK
ROWS_MAIN = 624

_mesh = plsc.VectorSubcoreMesh(
    core_axis_name="c", subcore_axis_name="s", num_cores=NC, num_subcores=NS)


def _gnn_body(ha, hb, ra0, rb0, ra1, rb1, src1, dst1, et1, outa, outb,
              rows0, rows1, rows2, relr0, relr1, relr2,
              sb0, sb1, sb2, eb0, eb1, eb2, db0, db1, db2,
              acc1, acc2, rel_sp0, rel_sp1,
              sgh0, sgh1, sgh2, sgr0, sgr1, sgr2,
              sse0, sse1, sse2, sd0, sd1, sd2, ss0, ss1, ss2):
  cid = lax.axis_index("c")
  sid = lax.axis_index("s")

  @pl.when(jnp.logical_and(cid == 0, sid == 0))
  def _():
    pltpu.sync_copy(ra0, rel_sp0)
    pltpu.sync_copy(ra1, rel_sp1)

  @pl.when(jnp.logical_and(cid == 1, sid == 0))
  def _():
    pltpu.sync_copy(rb0, rel_sp0)
    pltpu.sync_copy(rb1, rel_sp1)

  zv = jnp.zeros((16,), jnp.float32)

  @plsc.parallel_loop(0, CHUNK)
  def _(r):
    for c in range(DH // 16):
      rows0[r, pl.ds(c * 16, 16)] = zv

  row0 = sid * ROWS_MAIN
  tail = N_NODES - NS * ROWS_MAIN
  for acc in (acc1, acc2):
    for k in range(ROWS_MAIN // CHUNK):
      pltpu.sync_copy(rows0, acc.at[pl.ds(row0 + k * CHUNK, CHUNK)])
    rem = ROWS_MAIN % CHUNK
    pltpu.sync_copy(rows0.at[pl.ds(0, rem)],
                    acc.at[pl.ds(row0 + ROWS_MAIN - rem, rem)])

    @pl.when(sid == NS - 1)
    def _():
      pltpu.sync_copy(rows0.at[pl.ds(0, tail)],
                      acc.at[pl.ds(NS * ROWS_MAIN, tail)])

  plsc.subcore_barrier()

  base_e = sid * E_PER_T

  bufs = ((rows0, relr0, sb0, eb0, db0, sgh0, sgr0, sse0, sd0, ss0),
          (rows1, relr1, sb1, eb1, db1, sgh1, sgr1, sse1, sd1, ss1),
          (rows2, relr2, sb2, eb2, db2, sgh2, sgr2, sse2, sd2, ss2))

  def refill_se(i, b):
    sb, eb, sse = bufs[b][2], bufs[b][3], bufs[b][7]
    pltpu.async_copy(src1.at[pl.ds(base_e + i * CHUNK, CHUNK)], sb, sse)
    pltpu.async_copy(et1.at[pl.ds(base_e + i * CHUNK, CHUNK)], eb, sse)

  def wait_se(i, b):
    sb, eb, sse = bufs[b][2], bufs[b][3], bufs[b][7]
    pltpu.make_async_copy(src1.at[pl.ds(base_e + i * CHUNK, CHUNK)], sb,
                          sse).wait()
    pltpu.make_async_copy(et1.at[pl.ds(base_e + i * CHUNK, CHUNK)], eb,
                          sse).wait()

  def refill_d(i, b):
    db, sd = bufs[b][4], bufs[b][8]
    pltpu.async_copy(dst1.at[pl.ds(base_e + i * CHUNK, CHUNK)], db, sd)

  def wait_d(i, b):
    db, sd = bufs[b][4], bufs[b][8]
    pltpu.make_async_copy(dst1.at[pl.ds(base_e + i * CHUNK, CHUNK)], db,
                          sd).wait()

  def mul(b):
    rows, relr = bufs[b][0], bufs[b][1]

    @plsc.parallel_loop(0, CHUNK, unroll=4)
    def _(r):
      for c in range(DH // 16):
        s = pl.ds(c * 16, 16)
        rows[r, s] = rows[r, s] * relr[r, s]

  def make_phase(src_tab, rel_sp, acc):
    per_core = isinstance(src_tab, tuple)

    def issue_gather(b):
      rows, relr = bufs[b][0], bufs[b][1]
      sb, eb = bufs[b][2], bufs[b][3]
      sgh, sgr = bufs[b][5], bufs[b][6]
      if per_core:
        @pl.when(cid == 0)
        def _():
          pltpu.async_copy(src_tab[0].at[sb], rows, sgh)

        @pl.when(cid == 1)
        def _():
          pltpu.async_copy(src_tab[1].at[sb], rows, sgh)
      else:
        pltpu.async_copy(src_tab.at[sb], rows, sgh)
      pltpu.async_copy(rel_sp.at[eb], relr, sgr)

    def wait_gather(b):
      rows, relr = bufs[b][0], bufs[b][1]
      sb, eb = bufs[b][2], bufs[b][3]
      sgh, sgr = bufs[b][5], bufs[b][6]
      ref = src_tab[0] if per_core else src_tab
      pltpu.make_async_copy(ref.at[sb], rows, sgh).wait()
      pltpu.make_async_copy(rel_sp.at[eb], relr, sgr).wait()

    def issue_scatter(b):
      rows, db, ss = bufs[b][0], bufs[b][4], bufs[b][9]
      pltpu.async_copy(rows, acc.at[db], ss, add=True)

    def wait_scatter(b):
      rows, db, ss = bufs[b][0], bufs[b][4], bufs[b][9]
      pltpu.make_async_copy(rows, acc.at[db], ss).wait()

    def section(i, b, bn, bnn):
      @pl.when(i >= 2)
      def _():
        wait_scatter(bn)

      @pl.when(i + 1 < N_CHUNKS)
      def _():
        refill_d(i + 1, bn)
        wait_se(i + 1, bn)
        issue_gather(bn)

      @pl.when(i + 2 < N_CHUNKS)
      def _():
        refill_se(i + 2, bnn)

      wait_gather(b)
      mul(b)
      wait_d(i, b)
      issue_scatter(b)

    def run():
      refill_se(0, 0)
      refill_se(1, 1)
      refill_d(0, 0)
      wait_se(0, 0)
      issue_gather(0)

      def _trio(g, _):
        for k in range(3):
          section(3 * g + k, k, (k + 1) % 3, (k + 2) % 3)
        return _

      n_trios = N_CHUNKS // 3
      lax.fori_loop(0, n_trios, _trio, None)
      for i in range(3 * n_trios, N_CHUNKS):
        section(jnp.int32(i), i % 3, (i + 1) % 3, (i + 2) % 3)
      wait_scatter((N_CHUNKS - 2) % 3)
      wait_scatter((N_CHUNKS - 1) % 3)

    return run

  make_phase((ha, hb), rel_sp0, acc1)()
  plsc.subcore_barrier()
  make_phase(acc1, rel_sp1, acc2)()
  plsc.subcore_barrier()

  for c, out in ((0, outa), (1, outb)):

    @pl.when(cid == c)
    def _():
      pltpu.sync_copy(acc2.at[pl.ds(row0, ROWS_MAIN)],
                      out.at[pl.ds(row0, ROWS_MAIN)])

      @pl.when(sid == NS - 1)
      def _():
        pltpu.sync_copy(acc2.at[pl.ds(NS * ROWS_MAIN, tail)],
                        out.at[pl.ds(NS * ROWS_MAIN, tail)])


_gnn2 = pl.kernel(
    _gnn_body,
    out_type=(jax.ShapeDtypeStruct((N_NODES, DH), jnp.float32),
              jax.ShapeDtypeStruct((N_NODES, DH), jnp.float32)),
    mesh=_mesh,
    compiler_params=pltpu.CompilerParams(use_tc_tiling_on_sc=False),
    scratch_types=(
        [pltpu.VMEM((CHUNK, DH), jnp.float32)] * 6
        + [pltpu.VMEM((CHUNK,), jnp.int32)] * 9
        + [pltpu.VMEM_SHARED((N_NODES, DH), jnp.float32)] * 2
        + [pltpu.VMEM_SHARED((N_REL, DH), jnp.float32)] * 2
        + [pltpu.SemaphoreType.DMA] * 15
    ),
)


def kernel(node_features, edge_index, etype, rel_emb_0, rel_emb_1):
  src = edge_index[0].astype(jnp.int32)
  dst = edge_index[1].astype(jnp.int32)
  et = etype.astype(jnp.int32)
  ha = node_features[:, :DH]
  hb = node_features[:, DH:]
  oa, ob = _gnn2(ha, hb,
                 rel_emb_0[:, :DH], rel_emb_0[:, DH:],
                 rel_emb_1[:, :DH], rel_emb_1[:, DH:],
                 src, dst, et)
  return jnp.concatenate([oa, ob], axis=1)


# speedup vs baseline: 1.1675x; 1.0203x over previous
import jax
import jax.numpy as jnp
from jax import lax
from jax.experimental import pallas as pl
from jax.experimental.pallas import tpu as pltpu, tpu_sc as plsc

N_NODES = 10000
N_EDGES = 320000
D = 128
DH = D // 2
N_REL = 100

NC = 2
NS = 16
E_PER_T = N_EDGES // NS
CHUNK = 80
N_CHUNKS = E_PER_T // CHUNK
ROWS_MAIN = 624

_mesh = plsc.VectorSubcoreMesh(
    core_axis_name="c", subcore_axis_name="s", num_cores=NC, num_subcores=NS)


def _gnn_body(ha, hb, ra0, rb0, ra1, rb1, src1, dst1, et1, outa, outb,
              rows0, rows1, rows2, rows3, relr0, relr1, relr2, relr3,
              sb0, sb1, sb2, sb3, eb0, eb1, eb2, eb3, db0, db1, db2, db3,
              acc1, acc2, rel_sp0, rel_sp1,
              sgh0, sgh1, sgh2, sgh3, sgr0, sgr1, sgr2, sgr3,
              sse0, sse1, sse2, sse3, sd0, sd1, sd2, sd3,
              ss0, ss1, ss2, ss3):
  cid = lax.axis_index("c")
  sid = lax.axis_index("s")

  @pl.when(jnp.logical_and(cid == 0, sid == 0))
  def _():
    pltpu.sync_copy(ra0, rel_sp0)
    pltpu.sync_copy(ra1, rel_sp1)

  @pl.when(jnp.logical_and(cid == 1, sid == 0))
  def _():
    pltpu.sync_copy(rb0, rel_sp0)
    pltpu.sync_copy(rb1, rel_sp1)

  zv = jnp.zeros((16,), jnp.float32)

  @plsc.parallel_loop(0, CHUNK)
  def _(r):
    for c in range(DH // 16):
      rows0[r, pl.ds(c * 16, 16)] = zv

  row0 = sid * ROWS_MAIN
  tail = N_NODES - NS * ROWS_MAIN
  for acc in (acc1, acc2):
    for k in range(ROWS_MAIN // CHUNK):
      pltpu.sync_copy(rows0, acc.at[pl.ds(row0 + k * CHUNK, CHUNK)])
    rem = ROWS_MAIN % CHUNK
    pltpu.sync_copy(rows0.at[pl.ds(0, rem)],
                    acc.at[pl.ds(row0 + ROWS_MAIN - rem, rem)])

    @pl.when(sid == NS - 1)
    def _():
      pltpu.sync_copy(rows0.at[pl.ds(0, tail)],
                      acc.at[pl.ds(NS * ROWS_MAIN, tail)])

  plsc.subcore_barrier()

  base_e = sid * E_PER_T

  bufs = ((rows0, relr0, sb0, eb0, db0, sgh0, sgr0, sse0, sd0, ss0),
          (rows1, relr1, sb1, eb1, db1, sgh1, sgr1, sse1, sd1, ss1),
          (rows2, relr2, sb2, eb2, db2, sgh2, sgr2, sse2, sd2, ss2),
          (rows3, relr3, sb3, eb3, db3, sgh3, sgr3, sse3, sd3, ss3))

  def refill_se(i, b):
    sb, eb, sse = bufs[b][2], bufs[b][3], bufs[b][7]
    pltpu.async_copy(src1.at[pl.ds(base_e + i * CHUNK, CHUNK)], sb, sse)
    pltpu.async_copy(et1.at[pl.ds(base_e + i * CHUNK, CHUNK)], eb, sse)

  def wait_se(i, b):
    sb, eb, sse = bufs[b][2], bufs[b][3], bufs[b][7]
    pltpu.make_async_copy(src1.at[pl.ds(base_e + i * CHUNK, CHUNK)], sb,
                          sse).wait()
    pltpu.make_async_copy(et1.at[pl.ds(base_e + i * CHUNK, CHUNK)], eb,
                          sse).wait()

  def refill_d(i, b):
    db, sd = bufs[b][4], bufs[b][8]
    pltpu.async_copy(dst1.at[pl.ds(base_e + i * CHUNK, CHUNK)], db, sd)

  def wait_d(i, b):
    db, sd = bufs[b][4], bufs[b][8]
    pltpu.make_async_copy(dst1.at[pl.ds(base_e + i * CHUNK, CHUNK)], db,
                          sd).wait()

  def mul(b):
    rows, relr = bufs[b][0], bufs[b][1]

    @plsc.parallel_loop(0, CHUNK, unroll=4)
    def _(r):
      for c in range(DH // 16):
        s = pl.ds(c * 16, 16)
        rows[r, s] = rows[r, s] * relr[r, s]

  def make_phase(src_tab, rel_sp, acc):
    per_core = isinstance(src_tab, tuple)

    def issue_gather(b):
      rows, relr = bufs[b][0], bufs[b][1]
      sb, eb = bufs[b][2], bufs[b][3]
      sgh, sgr = bufs[b][5], bufs[b][6]
      if per_core:
        @pl.when(cid == 0)
        def _():
          pltpu.async_copy(src_tab[0].at[sb], rows, sgh)

        @pl.when(cid == 1)
        def _():
          pltpu.async_copy(src_tab[1].at[sb], rows, sgh)
      else:
        pltpu.async_copy(src_tab.at[sb], rows, sgh)
      pltpu.async_copy(rel_sp.at[eb], relr, sgr)

    def wait_gather(b):
      rows, relr = bufs[b][0], bufs[b][1]
      sb, eb = bufs[b][2], bufs[b][3]
      sgh, sgr = bufs[b][5], bufs[b][6]
      ref = src_tab[0] if per_core else src_tab
      pltpu.make_async_copy(ref.at[sb], rows, sgh).wait()
      pltpu.make_async_copy(rel_sp.at[eb], relr, sgr).wait()

    def issue_scatter(b):
      rows, db, ss = bufs[b][0], bufs[b][4], bufs[b][9]
      pltpu.async_copy(rows, acc.at[db], ss, add=True)

    def wait_scatter(b):
      rows, db, ss = bufs[b][0], bufs[b][4], bufs[b][9]
      pltpu.make_async_copy(rows, acc.at[db], ss).wait()

    def section(i, b, b2, b3):
      @pl.when(i >= 2)
      def _():
        wait_scatter(b2)

      @pl.when(i + 2 < N_CHUNKS)
      def _():
        refill_d(i + 2, b2)
        wait_se(i + 2, b2)
        issue_gather(b2)

      @pl.when(i + 3 < N_CHUNKS)
      def _():
        refill_se(i + 3, b3)

      wait_gather(b)
      mul(b)
      wait_d(i, b)
      issue_scatter(b)

    def run():
      refill_se(0, 0)
      refill_se(1, 1)
      refill_se(2, 2)
      refill_d(0, 0)
      refill_d(1, 1)
      wait_se(0, 0)
      issue_gather(0)
      wait_se(1, 1)
      issue_gather(1)

      def _quad(g, _):
        for k in range(4):
          section(4 * g + k, k, (k + 2) % 4, (k + 3) % 4)
        return _

      n_quads = N_CHUNKS // 4
      lax.fori_loop(0, n_quads, _quad, None)
      for i in range(4 * n_quads, N_CHUNKS):
        section(jnp.int32(i), i % 4, (i + 2) % 4, (i + 3) % 4)
      wait_scatter((N_CHUNKS - 2) % 4)
      wait_scatter((N_CHUNKS - 1) % 4)

    return run

  make_phase((ha, hb), rel_sp0, acc1)()
  plsc.subcore_barrier()
  make_phase(acc1, rel_sp1, acc2)()
  plsc.subcore_barrier()

  for c, out in ((0, outa), (1, outb)):

    @pl.when(cid == c)
    def _():
      pltpu.sync_copy(acc2.at[pl.ds(row0, ROWS_MAIN)],
                      out.at[pl.ds(row0, ROWS_MAIN)])

      @pl.when(sid == NS - 1)
      def _():
        pltpu.sync_copy(acc2.at[pl.ds(NS * ROWS_MAIN, tail)],
                        out.at[pl.ds(NS * ROWS_MAIN, tail)])


_gnn2 = pl.kernel(
    _gnn_body,
    out_type=(jax.ShapeDtypeStruct((N_NODES, DH), jnp.float32),
              jax.ShapeDtypeStruct((N_NODES, DH), jnp.float32)),
    mesh=_mesh,
    compiler_params=pltpu.CompilerParams(use_tc_tiling_on_sc=False),
    scratch_types=(
        [pltpu.VMEM((CHUNK, DH), jnp.float32)] * 8
        + [pltpu.VMEM((CHUNK,), jnp.int32)] * 12
        + [pltpu.VMEM_SHARED((N_NODES, DH), jnp.float32)] * 2
        + [pltpu.VMEM_SHARED((N_REL, DH), jnp.float32)] * 2
        + [pltpu.SemaphoreType.DMA] * 20
    ),
)


def kernel(node_features, edge_index, etype, rel_emb_0, rel_emb_1):
  src = edge_index[0].astype(jnp.int32)
  dst = edge_index[1].astype(jnp.int32)
  et = etype.astype(jnp.int32)
  ha = node_features[:, :DH]
  hb = node_features[:, DH:]
  oa, ob = _gnn2(ha, hb,
                 rel_emb_0[:, :DH], rel_emb_0[:, DH:],
                 rel_emb_1[:, :DH], rel_emb_1[:, DH:],
                 src, dst, et)
  return jnp.concatenate([oa, ob], axis=1)
